# baseline (speedup 1.0000x reference)
import jax
import jax.numpy as jnp
from jax.experimental import pallas as pl
from jax.experimental.pallas import tpu as pltpu

_K = 2048
_N = 2048
_NJ = 256
_NI = 1024
_J = _N // _NJ


def _lut(table, idx):
    r = idx.shape[0]
    t_lo = jnp.broadcast_to(table[:, :128], (r, 128))
    t_hi = jnp.broadcast_to(table[:, 128:], (r, 128))
    low = idx & 127
    lo = jnp.take_along_axis(t_lo, low, axis=1, mode="promise_in_bounds")
    hi = jnp.take_along_axis(t_hi, low, axis=1, mode="promise_in_bounds")
    return jnp.where(idx < 128, lo, hi)


def _make_fused(n_i):
    total = n_i * _J + 1

    def _fused(x_ref, idx_ref, wt_ref, bidx_ref, bt_ref, out_ref,
               wc_ref, xb_ref):
        s = pl.program_id(0)

        @pl.when(s < _J)
        def _dequant():
            w = _lut(wt_ref[...], idx_ref[...])
            wc_ref[pl.ds(s * _NJ, _NJ), :] = w.astype(jnp.bfloat16)

        @pl.when(jnp.logical_and(s % _J == 0, s < n_i * _J))
        def _cast():
            xb_ref[(s // _J) % 2] = x_ref[...].astype(jnp.bfloat16)

        @pl.when(s > 0)
        def _matmul():
            sm = s - 1
            i = sm // _J
            j = sm % _J
            xb = xb_ref[i % 2]
            wb = wc_ref[pl.ds(j * _NJ, _NJ), :]
            acc = jax.lax.dot_general(
                xb, wb, (((1,), (1,)), ((), ())),
                preferred_element_type=jnp.float32)
            bidx8 = jnp.broadcast_to(bidx_ref[0], (8, _NJ))
            bvec = _lut(bt_ref[...], bidx8)
            out_ref[...] = acc + bvec[0:1, :]

    return _fused, total


def kernel(input_, weight, weight_table, bias, bias_table):
    B, M0, K = input_.shape
    M = B * M0
    n_i = M // _NI
    x = input_.reshape(M, K)
    wt = weight_table.reshape(1, 256)
    bt = bias_table.reshape(1, 256)
    bidx = bias.reshape(_J, 1, _NJ)
    body, total = _make_fused(n_i)

    def _jm(s):
        return jnp.maximum(s - 1, 0) % _J

    out = pl.pallas_call(
        body,
        grid=(total,),
        in_specs=[
            pl.BlockSpec((_NI, _K), lambda s: (jnp.minimum(s // _J, n_i - 1), 0)),
            pl.BlockSpec((_NJ, _K), lambda s: (jnp.minimum(s, _J - 1), 0)),
            pl.BlockSpec((1, 256), lambda s: (0, 0)),
            pl.BlockSpec((1, 1, _NJ), lambda s: (_jm(s), 0, 0)),
            pl.BlockSpec((1, 256), lambda s: (0, 0)),
        ],
        out_specs=pl.BlockSpec(
            (_NI, _NJ),
            lambda s: (jnp.maximum(s - 1, 0) // _J, _jm(s))),
        out_shape=jax.ShapeDtypeStruct((M, _N), jnp.float32),
        scratch_shapes=[
            pltpu.VMEM((_N, _K), jnp.bfloat16),
            pltpu.VMEM((2, _NI, _K), jnp.bfloat16),
        ],
    )(x, weight, wt, bidx, bt)
    return out.reshape(B, M0, _N)
